# baseline (speedup 1.0000x reference)
import functools

import jax
import jax.numpy as jnp
from jax import lax
from jax.experimental import pallas as pl
from jax.experimental.pallas import tpu as pltpu
from jax.experimental.pallas import tpu_sc as plsc

N0, N1 = 64, 64
D0, D1 = 1024, 1024
NC, NS = 2, 16
NW = NC * NS
B = 256
BPW = B // NW
REP = 32

_mesh = plsc.VectorSubcoreMesh(core_axis_name="c", subcore_axis_name="s")


@functools.partial(
    pl.kernel,
    mesh=_mesh,
    out_type=jax.ShapeDtypeStruct((B * N0, D0 + D1), jnp.float32),
    scratch_types=[
        pltpu.VMEM((BPW, REP), jnp.int32),
        pltpu.VMEM((N0, D0), jnp.float32),
        pltpu.VMEM((REP, D1), jnp.float32),
        pltpu.SemaphoreType.DMA,
    ],
)
def _sc_kernel(idx_hbm, w0_hbm, w1_hbm, out_hbm, idx_v, w0_v, rep_v, sem):
    wid = lax.axis_index("s") * NC + lax.axis_index("c")
    base = wid * BPW
    pltpu.sync_copy(idx_hbm.at[pl.ds(base, BPW)], idx_v)
    pltpu.sync_copy(w0_hbm, w0_v)
    for b in range(BPW):
        row0 = (base + b) * N0
        pltpu.async_copy(w1_hbm.at[idx_v.at[b]], rep_v, sem).wait()
        pltpu.sync_copy(rep_v, out_hbm.at[pl.ds(row0, REP), pl.ds(D0, D1)])
        pltpu.sync_copy(rep_v, out_hbm.at[pl.ds(row0 + REP, REP), pl.ds(D0, D1)])
        pltpu.sync_copy(w0_v, out_hbm.at[pl.ds(row0, N0), pl.ds(0, D0)])


def kernel(position_ids, w0, w1):
    pid = position_ids.reshape(-1).astype(jnp.int32)
    idx_rep = jnp.broadcast_to(pid[:, None], (B, REP)).reshape(B // BPW, BPW, REP)
    out = _sc_kernel(
        idx_rep.reshape(B, REP), w0.reshape(N0, D0), w1.reshape(N1, D1)
    )
    return out.reshape(*position_ids.shape, N0, D0 + D1)
